# Initial kernel scaffold; baseline (speedup 1.0000x reference)
import jax
import jax.numpy as jnp
from jax.experimental import pallas as pl


def kernel(x, b_pre, W_enc, W_dec, b_post):
    raise NotImplementedError("write your pallas kernel here")


# speedup vs baseline: 1.6072x; 1.6072x over previous
import functools
import math

import jax
import jax.numpy as jnp
from jax import lax
from jax.experimental import pallas as pl
from jax.experimental.pallas import tpu as pltpu
from jax.experimental.pallas import tpu_sc as plsc

D_MODEL = 2048
N_FEATURES = 32768
K = 32
BATCH = 2048
SQRT_D = math.sqrt(D_MODEL)

NSEG = 8
SEG = N_FEATURES // NSEG
LANES = 128
FB = 512
TPS = FB // LANES
STEPS_PER_SEG = SEG // FB
NSTEPS = N_FEATURES // FB
NCAND = NSEG * LANES
RB = 256


def _prep_body(x_ref, bpre_ref, xn_ref, xm_ref):
    xn = x_ref[...] * jnp.float32(SQRT_D)
    xn_ref[...] = xn
    xm_ref[...] = (xn - bpre_ref[...]).astype(jnp.bfloat16)


def _prep(x, b_pre2d):
    return pl.pallas_call(
        _prep_body,
        grid=(BATCH // RB,),
        in_specs=[
            pl.BlockSpec((RB, D_MODEL), lambda i: (i, 0)),
            pl.BlockSpec((1, D_MODEL), lambda i: (0, 0)),
        ],
        out_specs=[
            pl.BlockSpec((RB, D_MODEL), lambda i: (i, 0)),
            pl.BlockSpec((RB, D_MODEL), lambda i: (i, 0)),
        ],
        out_shape=[
            jax.ShapeDtypeStruct((BATCH, D_MODEL), jnp.float32),
            jax.ShapeDtypeStruct((BATCH, D_MODEL), jnp.bfloat16),
        ],
    )(x, b_pre2d)


def _enc_body(xm_ref, w_ref, av_ref, aj_ref):
    j = pl.program_id(0)
    wb = w_ref[...].astype(jnp.bfloat16)
    prod = jnp.dot(xm_ref[...], wb, preferred_element_type=jnp.float32)
    b4 = (j % STEPS_PER_SEG) * TPS
    c0 = prod[:, 0:128]
    c1 = prod[:, 128:256]
    c2 = prod[:, 256:384]
    c3 = prod[:, 384:512]
    i0, i1, i2, i3 = (jnp.int32(0), jnp.int32(1), jnp.int32(2), jnp.int32(3))
    m01 = jnp.maximum(c0, c1)
    s01 = jnp.where(c0 >= c1, i0, i1)
    m23 = jnp.maximum(c2, c3)
    s23 = jnp.where(c2 >= c3, i2, i3)
    lv = jnp.maximum(m01, m23)
    lj = jnp.where(m01 >= m23, s01, s23) + b4

    first = (j % STEPS_PER_SEG) == 0

    @pl.when(first)
    def _():
        av_ref[0] = lv
        aj_ref[0] = lj

    @pl.when(jnp.logical_not(first))
    def _():
        cur = av_ref[0]
        better = lv > cur
        av_ref[0] = jnp.where(better, lv, cur)
        aj_ref[0] = jnp.where(better, lj, aj_ref[0])


def _encode_binmax(xm, W_enc):
    return pl.pallas_call(
        _enc_body,
        grid=(NSTEPS,),
        in_specs=[
            pl.BlockSpec((BATCH, D_MODEL), lambda j: (0, 0)),
            pl.BlockSpec((D_MODEL, FB), lambda j: (0, j)),
        ],
        out_specs=[
            pl.BlockSpec((1, BATCH, LANES), lambda j: (j // STEPS_PER_SEG, 0, 0)),
            pl.BlockSpec((1, BATCH, LANES), lambda j: (j // STEPS_PER_SEG, 0, 0)),
        ],
        out_shape=[
            jax.ShapeDtypeStruct((NSEG, BATCH, LANES), jnp.float32),
            jax.ShapeDtypeStruct((NSEG, BATCH, LANES), jnp.int32),
        ],
        compiler_params=pltpu.CompilerParams(
            dimension_semantics=("arbitrary",),
        ),
    )(xm, W_enc)


def _topk_body(av_ref, aj_ref, w_ref, i_ref):
    v = av_ref[...]
    jj = aj_ref[...]
    seg = lax.broadcasted_iota(jnp.int32, v.shape, 0)
    lane = lax.broadcasted_iota(jnp.int32, v.shape, 2)
    idxarr = seg * SEG + jj * LANES + lane
    posarr = seg * LANES + lane
    neg = jnp.float32(-jnp.inf)
    ws, isel = [], []
    for _ in range(K):
        m = jnp.max(jnp.max(v, axis=0), axis=1, keepdims=True)
        eq = v == m[None, :, :]
        pos = jnp.where(eq, posarr, jnp.int32(NCAND))
        p = jnp.min(jnp.min(pos, axis=0), axis=1, keepdims=True)
        fm = pos == p[None, :, :]
        ik = jnp.sum(jnp.sum(jnp.where(fm, idxarr, 0), axis=0),
                     axis=1, keepdims=True)
        v = jnp.where(fm, neg, v)
        ws.append(m)
        isel.append(ik)
    w_ref[...] = jnp.concatenate(ws, axis=1)
    i_ref[...] = jnp.concatenate(isel, axis=1)


def _topk(av, aj):
    return pl.pallas_call(
        _topk_body,
        grid=(BATCH // RB,),
        in_specs=[
            pl.BlockSpec((NSEG, RB, LANES), lambda i: (0, i, 0)),
            pl.BlockSpec((NSEG, RB, LANES), lambda i: (0, i, 0)),
        ],
        out_specs=[
            pl.BlockSpec((RB, K), lambda i: (i, 0)),
            pl.BlockSpec((RB, K), lambda i: (i, 0)),
        ],
        out_shape=[
            jax.ShapeDtypeStruct((BATCH, K), jnp.float32),
            jax.ShapeDtypeStruct((BATCH, K), jnp.int32),
        ],
    )(av, aj)


NW = 32
RPW = BATCH // NW
CH = 16
NCH = K // CH
NV = D_MODEL // 16


def _splat(val):
    return jnp.full((16,), val, jnp.int32)


def _dec_body(idx_hbm, w_hbm, wdec_hbm, bpost_hbm, y_hbm,
              idx_v, w_v, bpost_v, gbuf0, gbuf1, out_v, sem0, sem1):
    cid = lax.axis_index("c")
    sid = lax.axis_index("s")
    wid = sid * 2 + cid
    base = wid * RPW
    pltpu.sync_copy(idx_hbm.at[pl.ds(base, RPW)], idx_v)
    pltpu.sync_copy(w_hbm.at[pl.ds(base * K, RPW * K)], w_v)
    pltpu.sync_copy(bpost_hbm, bpost_v)

    bufs = (gbuf0, gbuf1)
    sems = (sem0, sem1)

    def start(r, c, slot):
        ivec = idx_v[r, pl.ds(c * CH, CH)]
        pltpu.async_copy(wdec_hbm.at[ivec], bufs[slot], sems[slot])

    def wait(r, c, slot):
        ivec = idx_v[r, pl.ds(c * CH, CH)]
        pltpu.make_async_copy(wdec_hbm.at[ivec], bufs[slot], sems[slot]).wait()

    def compute(r, c, slot):
        buf = bufs[slot]
        wrow = w_v[pl.ds(r * K + c * CH, CH)]
        wvecs = [jnp.broadcast_to(wrow[kk], (16,)) for kk in range(CH)]

        def vbody(vi, _):
            sl = pl.ds(vi * 16, 16)
            if c == 0:
                acc = bpost_v[sl]
            else:
                acc = out_v[0, sl]
            for kk in range(CH):
                acc = acc + wvecs[kk] * buf[kk, sl]
            out_v[0, sl] = acc
            return 0

        lax.fori_loop(0, NV, vbody, 0)

    start(0, 0, 0)

    def row_body(r, _):
        wait(r, 0, 0)
        start(r, 1, 1)
        compute(r, 0, 0)
        wait(r, 1, 1)

        @pl.when(r < RPW - 1)
        def _():
            start(r + 1, 0, 0)

        compute(r, 1, 1)
        pltpu.sync_copy(out_v, y_hbm.at[pl.ds(base + r, 1)])
        return 0

    lax.fori_loop(0, RPW, row_body, 0)


def _decode(indices, weights, W_dec, b_post):
    mesh = plsc.VectorSubcoreMesh(core_axis_name="c", subcore_axis_name="s")
    f = functools.partial(
        pl.kernel,
        mesh=mesh,
        out_type=jax.ShapeDtypeStruct((BATCH, D_MODEL), jnp.float32),
        scratch_types=[
            pltpu.VMEM((RPW, K), jnp.int32),
            pltpu.VMEM((RPW * K,), jnp.float32),
            pltpu.VMEM((D_MODEL,), jnp.float32),
            pltpu.VMEM((CH, D_MODEL), jnp.float32),
            pltpu.VMEM((CH, D_MODEL), jnp.float32),
            pltpu.VMEM((1, D_MODEL), jnp.float32),
            pltpu.SemaphoreType.DMA,
            pltpu.SemaphoreType.DMA,
        ],
    )(_dec_body)
    return f(indices, weights.reshape(BATCH * K), W_dec, b_post)


def _loss_body(xn_ref, yn_ref, y_ref, rl_ref, ll_ref):
    xn = xn_ref[...]
    yn = yn_ref[...]
    y_ref[...] = yn / jnp.float32(SQRT_D)
    d = xn - yn
    r = jnp.sum(d * d, axis=1, keepdims=True) * jnp.float32(1.0 / D_MODEL)
    rl_ref[...] = r
    ll_ref[...] = r + jnp.float32(1.0 / 32.0) * r


def _losses(xn, yn):
    return pl.pallas_call(
        _loss_body,
        grid=(BATCH // RB,),
        in_specs=[
            pl.BlockSpec((RB, D_MODEL), lambda i: (i, 0)),
            pl.BlockSpec((RB, D_MODEL), lambda i: (i, 0)),
        ],
        out_specs=[
            pl.BlockSpec((RB, D_MODEL), lambda i: (i, 0)),
            pl.BlockSpec((RB, 1), lambda i: (i, 0)),
            pl.BlockSpec((RB, 1), lambda i: (i, 0)),
        ],
        out_shape=[
            jax.ShapeDtypeStruct((BATCH, D_MODEL), jnp.float32),
            jax.ShapeDtypeStruct((BATCH, 1), jnp.float32),
            jax.ShapeDtypeStruct((BATCH, 1), jnp.float32),
        ],
    )(xn, yn)


def kernel(x, b_pre, W_enc, W_dec, b_post):
    xn, xm = _prep(x, b_pre.reshape(1, D_MODEL))
    av, aj = _encode_binmax(xm, W_enc)
    weights, indices = _topk(av, aj)
    y_normed = _decode(indices, weights, W_dec, b_post)
    y, rl, ll = _losses(xn, y_normed)
    recon_loss = rl.reshape(BATCH)
    loss = ll.reshape(BATCH)
    return (xn, x, weights, indices, y_normed, y, recon_loss, recon_loss, loss)


# speedup vs baseline: 1.6191x; 1.0074x over previous
import functools
import math

import jax
import jax.numpy as jnp
from jax import lax
from jax.experimental import pallas as pl
from jax.experimental.pallas import tpu as pltpu
from jax.experimental.pallas import tpu_sc as plsc

D_MODEL = 2048
N_FEATURES = 32768
K = 32
BATCH = 2048
SQRT_D = math.sqrt(D_MODEL)

NSEG = 8
SEG = N_FEATURES // NSEG
LANES = 128
FB = 512
TPS = FB // LANES
STEPS_PER_SEG = SEG // FB
NSTEPS = N_FEATURES // FB
NCAND = NSEG * LANES
RB = 256


def _prep_body(x_ref, bpre_ref, xn_ref, xm_ref):
    xn = x_ref[...] * jnp.float32(SQRT_D)
    xn_ref[...] = xn
    xm_ref[...] = (xn - bpre_ref[...]).astype(jnp.bfloat16)


def _prep(x, b_pre2d):
    return pl.pallas_call(
        _prep_body,
        grid=(BATCH // RB,),
        in_specs=[
            pl.BlockSpec((RB, D_MODEL), lambda i: (i, 0)),
            pl.BlockSpec((1, D_MODEL), lambda i: (0, 0)),
        ],
        out_specs=[
            pl.BlockSpec((RB, D_MODEL), lambda i: (i, 0)),
            pl.BlockSpec((RB, D_MODEL), lambda i: (i, 0)),
        ],
        out_shape=[
            jax.ShapeDtypeStruct((BATCH, D_MODEL), jnp.float32),
            jax.ShapeDtypeStruct((BATCH, D_MODEL), jnp.bfloat16),
        ],
    )(x, b_pre2d)


def _enc_body(xm_ref, w_ref, av_ref, aj_ref):
    j = pl.program_id(0)
    wb = w_ref[...].astype(jnp.bfloat16)
    prod = jnp.dot(xm_ref[...], wb, preferred_element_type=jnp.float32)
    b4 = (j % STEPS_PER_SEG) * TPS
    c0 = prod[:, 0:128]
    c1 = prod[:, 128:256]
    c2 = prod[:, 256:384]
    c3 = prod[:, 384:512]
    i0, i1, i2, i3 = (jnp.int32(0), jnp.int32(1), jnp.int32(2), jnp.int32(3))
    m01 = jnp.maximum(c0, c1)
    s01 = jnp.where(c0 >= c1, i0, i1)
    m23 = jnp.maximum(c2, c3)
    s23 = jnp.where(c2 >= c3, i2, i3)
    lv = jnp.maximum(m01, m23)
    lj = jnp.where(m01 >= m23, s01, s23) + b4

    first = (j % STEPS_PER_SEG) == 0

    @pl.when(first)
    def _():
        av_ref[0] = lv
        aj_ref[0] = lj

    @pl.when(jnp.logical_not(first))
    def _():
        cur = av_ref[0]
        better = lv > cur
        av_ref[0] = jnp.where(better, lv, cur)
        aj_ref[0] = jnp.where(better, lj, aj_ref[0])


def _encode_binmax(xm, W_enc):
    return pl.pallas_call(
        _enc_body,
        grid=(NSTEPS,),
        in_specs=[
            pl.BlockSpec((BATCH, D_MODEL), lambda j: (0, 0)),
            pl.BlockSpec((D_MODEL, FB), lambda j: (0, j)),
        ],
        out_specs=[
            pl.BlockSpec((1, BATCH, LANES), lambda j: (j // STEPS_PER_SEG, 0, 0)),
            pl.BlockSpec((1, BATCH, LANES), lambda j: (j // STEPS_PER_SEG, 0, 0)),
        ],
        out_shape=[
            jax.ShapeDtypeStruct((NSEG, BATCH, LANES), jnp.float32),
            jax.ShapeDtypeStruct((NSEG, BATCH, LANES), jnp.int32),
        ],
        compiler_params=pltpu.CompilerParams(
            dimension_semantics=("arbitrary",),
        ),
    )(xm, W_enc)


def _topk_body(av_ref, aj_ref, w_ref, i_ref):
    v = av_ref[...]
    jj = aj_ref[...]
    seg = lax.broadcasted_iota(jnp.int32, v.shape, 0)
    lane = lax.broadcasted_iota(jnp.int32, v.shape, 2)
    idxarr = seg * SEG + jj * LANES + lane
    posarr = seg * LANES + lane
    neg = jnp.float32(-jnp.inf)
    ws, isel = [], []
    for _ in range(K):
        m = jnp.max(jnp.max(v, axis=0), axis=1, keepdims=True)
        eq = v == m[None, :, :]
        pos = jnp.where(eq, posarr, jnp.int32(NCAND))
        p = jnp.min(jnp.min(pos, axis=0), axis=1, keepdims=True)
        fm = pos == p[None, :, :]
        ik = jnp.sum(jnp.sum(jnp.where(fm, idxarr, 0), axis=0),
                     axis=1, keepdims=True)
        v = jnp.where(fm, neg, v)
        ws.append(m)
        isel.append(ik)
    w_ref[...] = jnp.concatenate(ws, axis=1)
    i_ref[...] = jnp.concatenate(isel, axis=1)


def _topk(av, aj):
    return pl.pallas_call(
        _topk_body,
        grid=(BATCH // RB,),
        in_specs=[
            pl.BlockSpec((NSEG, RB, LANES), lambda i: (0, i, 0)),
            pl.BlockSpec((NSEG, RB, LANES), lambda i: (0, i, 0)),
        ],
        out_specs=[
            pl.BlockSpec((RB, K), lambda i: (i, 0)),
            pl.BlockSpec((RB, K), lambda i: (i, 0)),
        ],
        out_shape=[
            jax.ShapeDtypeStruct((BATCH, K), jnp.float32),
            jax.ShapeDtypeStruct((BATCH, K), jnp.int32),
        ],
    )(av, aj)


NW = 32
RPW = BATCH // NW
CH = 16
NCH = K // CH
NV = D_MODEL // 16


def _splat(val):
    return jnp.full((16,), val, jnp.int32)


def _dec_body(idx_hbm, w_hbm, wdec_hbm, bpost_hbm, y_hbm,
              idx_v, w_v, bpost_v, gbuf0, gbuf1, out_v, sem0, sem1):
    cid = lax.axis_index("c")
    sid = lax.axis_index("s")
    wid = sid * 2 + cid
    base = wid * RPW
    pltpu.sync_copy(idx_hbm.at[pl.ds(base, RPW)], idx_v)
    pltpu.sync_copy(w_hbm.at[pl.ds(base * K, RPW * K)], w_v)
    pltpu.sync_copy(bpost_hbm, bpost_v)

    bufs = (gbuf0, gbuf1)
    sems = (sem0, sem1)

    def start(r, c, slot):
        ivec = idx_v[r, pl.ds(c * CH, CH)]
        pltpu.async_copy(wdec_hbm.at[ivec], bufs[slot], sems[slot])

    def wait(r, c, slot):
        ivec = idx_v[r, pl.ds(c * CH, CH)]
        pltpu.make_async_copy(wdec_hbm.at[ivec], bufs[slot], sems[slot]).wait()

    def compute(r, c, slot):
        buf = bufs[slot]
        wrow = w_v[pl.ds(r * K + c * CH, CH)]
        wvecs = [jnp.broadcast_to(wrow[kk], (16,)) for kk in range(CH)]

        def vbody(vi, _):
            for u in range(4):
                sl = pl.ds(vi * 64 + u * 16, 16)
                if c == 0:
                    acc = bpost_v[sl]
                else:
                    acc = out_v[0, sl]
                for kk in range(CH):
                    acc = acc + wvecs[kk] * buf[kk, sl]
                out_v[0, sl] = acc
            return 0

        lax.fori_loop(0, NV // 4, vbody, 0)

    start(0, 0, 0)

    def row_body(r, _):
        wait(r, 0, 0)
        start(r, 1, 1)
        compute(r, 0, 0)
        wait(r, 1, 1)

        @pl.when(r < RPW - 1)
        def _():
            start(r + 1, 0, 0)

        compute(r, 1, 1)
        pltpu.sync_copy(out_v, y_hbm.at[pl.ds(base + r, 1)])
        return 0

    lax.fori_loop(0, RPW, row_body, 0)


def _decode(indices, weights, W_dec, b_post):
    mesh = plsc.VectorSubcoreMesh(core_axis_name="c", subcore_axis_name="s")
    f = functools.partial(
        pl.kernel,
        mesh=mesh,
        out_type=jax.ShapeDtypeStruct((BATCH, D_MODEL), jnp.float32),
        scratch_types=[
            pltpu.VMEM((RPW, K), jnp.int32),
            pltpu.VMEM((RPW * K,), jnp.float32),
            pltpu.VMEM((D_MODEL,), jnp.float32),
            pltpu.VMEM((CH, D_MODEL), jnp.float32),
            pltpu.VMEM((CH, D_MODEL), jnp.float32),
            pltpu.VMEM((1, D_MODEL), jnp.float32),
            pltpu.SemaphoreType.DMA,
            pltpu.SemaphoreType.DMA,
        ],
    )(_dec_body)
    return f(indices, weights.reshape(BATCH * K), W_dec, b_post)


def _loss_body(xn_ref, yn_ref, y_ref, rl_ref, ll_ref):
    xn = xn_ref[...]
    yn = yn_ref[...]
    y_ref[...] = yn / jnp.float32(SQRT_D)
    d = xn - yn
    r = jnp.sum(d * d, axis=1, keepdims=True) * jnp.float32(1.0 / D_MODEL)
    rl_ref[...] = r
    ll_ref[...] = r + jnp.float32(1.0 / 32.0) * r


def _losses(xn, yn):
    return pl.pallas_call(
        _loss_body,
        grid=(BATCH // RB,),
        in_specs=[
            pl.BlockSpec((RB, D_MODEL), lambda i: (i, 0)),
            pl.BlockSpec((RB, D_MODEL), lambda i: (i, 0)),
        ],
        out_specs=[
            pl.BlockSpec((RB, D_MODEL), lambda i: (i, 0)),
            pl.BlockSpec((RB, 1), lambda i: (i, 0)),
            pl.BlockSpec((RB, 1), lambda i: (i, 0)),
        ],
        out_shape=[
            jax.ShapeDtypeStruct((BATCH, D_MODEL), jnp.float32),
            jax.ShapeDtypeStruct((BATCH, 1), jnp.float32),
            jax.ShapeDtypeStruct((BATCH, 1), jnp.float32),
        ],
    )(xn, yn)


def kernel(x, b_pre, W_enc, W_dec, b_post):
    xn, xm = _prep(x, b_pre.reshape(1, D_MODEL))
    av, aj = _encode_binmax(xm, W_enc)
    weights, indices = _topk(av, aj)
    y_normed = _decode(indices, weights, W_dec, b_post)
    y, rl, ll = _losses(xn, y_normed)
    recon_loss = rl.reshape(BATCH)
    loss = ll.reshape(BATCH)
    return (xn, x, weights, indices, y_normed, y, recon_loss, recon_loss, loss)


# speedup vs baseline: 1.6948x; 1.0468x over previous
import functools
import math

import jax
import jax.numpy as jnp
from jax import lax
from jax.experimental import pallas as pl
from jax.experimental.pallas import tpu as pltpu
from jax.experimental.pallas import tpu_sc as plsc

D_MODEL = 2048
N_FEATURES = 32768
K = 32
BATCH = 2048
SQRT_D = math.sqrt(D_MODEL)

NSEG = 8
SEG = N_FEATURES // NSEG
LANES = 128
FB = 512
TPS = FB // LANES
STEPS_PER_SEG = SEG // FB
NSTEPS = N_FEATURES // FB
NCAND = NSEG * LANES
RB = 256


def _prep_body(x_ref, bpre_ref, xn_ref, xm_ref):
    xn = x_ref[...] * jnp.float32(SQRT_D)
    xn_ref[...] = xn
    xm_ref[...] = (xn - bpre_ref[...]).astype(jnp.bfloat16)


def _prep(x, b_pre2d):
    return pl.pallas_call(
        _prep_body,
        grid=(BATCH // RB,),
        in_specs=[
            pl.BlockSpec((RB, D_MODEL), lambda i: (i, 0)),
            pl.BlockSpec((1, D_MODEL), lambda i: (0, 0)),
        ],
        out_specs=[
            pl.BlockSpec((RB, D_MODEL), lambda i: (i, 0)),
            pl.BlockSpec((RB, D_MODEL), lambda i: (i, 0)),
        ],
        out_shape=[
            jax.ShapeDtypeStruct((BATCH, D_MODEL), jnp.float32),
            jax.ShapeDtypeStruct((BATCH, D_MODEL), jnp.bfloat16),
        ],
    )(x, b_pre2d)


def _enc_body(xm_ref, w_ref, av_ref, aj_ref):
    j = pl.program_id(0)
    wb = w_ref[...].astype(jnp.bfloat16)
    prod = jnp.dot(xm_ref[...], wb, preferred_element_type=jnp.float32)
    b4 = (j % STEPS_PER_SEG) * TPS
    c0 = prod[:, 0:128]
    c1 = prod[:, 128:256]
    c2 = prod[:, 256:384]
    c3 = prod[:, 384:512]
    i0, i1, i2, i3 = (jnp.int32(0), jnp.int32(1), jnp.int32(2), jnp.int32(3))
    m01 = jnp.maximum(c0, c1)
    s01 = jnp.where(c0 >= c1, i0, i1)
    m23 = jnp.maximum(c2, c3)
    s23 = jnp.where(c2 >= c3, i2, i3)
    lv = jnp.maximum(m01, m23)
    lj = jnp.where(m01 >= m23, s01, s23) + b4

    first = (j % STEPS_PER_SEG) == 0

    @pl.when(first)
    def _():
        av_ref[0] = lv
        aj_ref[0] = lj

    @pl.when(jnp.logical_not(first))
    def _():
        cur = av_ref[0]
        better = lv > cur
        av_ref[0] = jnp.where(better, lv, cur)
        aj_ref[0] = jnp.where(better, lj, aj_ref[0])


def _encode_binmax(xm, W_enc):
    return pl.pallas_call(
        _enc_body,
        grid=(NSTEPS,),
        in_specs=[
            pl.BlockSpec((BATCH, D_MODEL), lambda j: (0, 0)),
            pl.BlockSpec((D_MODEL, FB), lambda j: (0, j)),
        ],
        out_specs=[
            pl.BlockSpec((1, BATCH, LANES), lambda j: (j // STEPS_PER_SEG, 0, 0)),
            pl.BlockSpec((1, BATCH, LANES), lambda j: (j // STEPS_PER_SEG, 0, 0)),
        ],
        out_shape=[
            jax.ShapeDtypeStruct((NSEG, BATCH, LANES), jnp.float32),
            jax.ShapeDtypeStruct((NSEG, BATCH, LANES), jnp.int32),
        ],
        compiler_params=pltpu.CompilerParams(
            dimension_semantics=("arbitrary",),
        ),
    )(xm, W_enc)


def _topk_body(av_ref, aj_ref, w_ref, i_ref):
    v = av_ref[...]
    jj = aj_ref[...]
    seg = lax.broadcasted_iota(jnp.int32, v.shape, 0)
    lane = lax.broadcasted_iota(jnp.int32, v.shape, 2)
    idxarr = seg * SEG + jj * LANES + lane
    posarr = seg * LANES + lane
    neg = jnp.float32(-jnp.inf)
    ws, isel = [], []
    for _ in range(K):
        m = jnp.max(jnp.max(v, axis=0), axis=1, keepdims=True)
        eq = v == m[None, :, :]
        pos = jnp.where(eq, posarr, jnp.int32(NCAND))
        p = jnp.min(jnp.min(pos, axis=0), axis=1, keepdims=True)
        fm = pos == p[None, :, :]
        ik = jnp.sum(jnp.sum(jnp.where(fm, idxarr, 0), axis=0),
                     axis=1, keepdims=True)
        v = jnp.where(fm, neg, v)
        ws.append(m)
        isel.append(ik)
    w_ref[...] = jnp.concatenate(ws, axis=1)
    i_ref[...] = jnp.concatenate(isel, axis=1)


NQ = 4
BQ = BATCH // NQ


def _topk(av, aj, qi):
    qb = qi * (BQ // RB)
    return pl.pallas_call(
        _topk_body,
        grid=(BQ // RB,),
        in_specs=[
            pl.BlockSpec((NSEG, RB, LANES), lambda i: (0, qb + i, 0)),
            pl.BlockSpec((NSEG, RB, LANES), lambda i: (0, qb + i, 0)),
        ],
        out_specs=[
            pl.BlockSpec((RB, K), lambda i: (i, 0)),
            pl.BlockSpec((RB, K), lambda i: (i, 0)),
        ],
        out_shape=[
            jax.ShapeDtypeStruct((BQ, K), jnp.float32),
            jax.ShapeDtypeStruct((BQ, K), jnp.int32),
        ],
    )(av, aj)


NW = 32
RPW = BQ // NW
CH = 16
NCH = K // CH
NV = D_MODEL // 16


def _splat(val):
    return jnp.full((16,), val, jnp.int32)


def _dec_body(idx_hbm, w_hbm, wdec_hbm, bpost_hbm, y_hbm,
              idx_v, w_v, bpost_v, gbuf0, gbuf1, out_v, sem0, sem1):
    cid = lax.axis_index("c")
    sid = lax.axis_index("s")
    wid = sid * 2 + cid
    base = wid * RPW
    pltpu.sync_copy(idx_hbm.at[pl.ds(base, RPW)], idx_v)
    pltpu.sync_copy(w_hbm.at[pl.ds(base * K, RPW * K)], w_v)
    pltpu.sync_copy(bpost_hbm, bpost_v)

    bufs = (gbuf0, gbuf1)
    sems = (sem0, sem1)

    def start(r, c, slot):
        ivec = idx_v[r, pl.ds(c * CH, CH)]
        pltpu.async_copy(wdec_hbm.at[ivec], bufs[slot], sems[slot])

    def wait(r, c, slot):
        ivec = idx_v[r, pl.ds(c * CH, CH)]
        pltpu.make_async_copy(wdec_hbm.at[ivec], bufs[slot], sems[slot]).wait()

    def compute(r, c, slot):
        buf = bufs[slot]
        wrow = w_v[pl.ds(r * K + c * CH, CH)]
        wvecs = [jnp.broadcast_to(wrow[kk], (16,)) for kk in range(CH)]

        def vbody(vi, _):
            for u in range(4):
                sl = pl.ds(vi * 64 + u * 16, 16)
                if c == 0:
                    acc = bpost_v[sl]
                else:
                    acc = out_v[0, sl]
                for kk in range(CH):
                    acc = acc + wvecs[kk] * buf[kk, sl]
                out_v[0, sl] = acc
            return 0

        lax.fori_loop(0, NV // 4, vbody, 0)

    start(0, 0, 0)

    def row_body(r, _):
        wait(r, 0, 0)
        start(r, 1, 1)
        compute(r, 0, 0)
        wait(r, 1, 1)

        @pl.when(r < RPW - 1)
        def _():
            start(r + 1, 0, 0)

        compute(r, 1, 1)
        pltpu.sync_copy(out_v, y_hbm.at[pl.ds(base + r, 1)])
        return 0

    lax.fori_loop(0, RPW, row_body, 0)


def _decode(indices, weights, W_dec, b_post):
    mesh = plsc.VectorSubcoreMesh(core_axis_name="c", subcore_axis_name="s")
    f = functools.partial(
        pl.kernel,
        mesh=mesh,
        out_type=jax.ShapeDtypeStruct((BQ, D_MODEL), jnp.float32),
        scratch_types=[
            pltpu.VMEM((RPW, K), jnp.int32),
            pltpu.VMEM((RPW * K,), jnp.float32),
            pltpu.VMEM((D_MODEL,), jnp.float32),
            pltpu.VMEM((CH, D_MODEL), jnp.float32),
            pltpu.VMEM((CH, D_MODEL), jnp.float32),
            pltpu.VMEM((1, D_MODEL), jnp.float32),
            pltpu.SemaphoreType.DMA,
            pltpu.SemaphoreType.DMA,
        ],
    )(_dec_body)
    return f(indices, weights.reshape(BQ * K), W_dec, b_post)


def _loss_body(xn_ref, yn_ref, y_ref, rl_ref, ll_ref):
    xn = xn_ref[...]
    yn = yn_ref[...]
    y_ref[...] = yn / jnp.float32(SQRT_D)
    d = xn - yn
    r = jnp.sum(d * d, axis=1, keepdims=True) * jnp.float32(1.0 / D_MODEL)
    rl_ref[...] = r
    ll_ref[...] = r + jnp.float32(1.0 / 32.0) * r


def _losses(xn, yn, qi):
    qb = qi * (BQ // RB)
    return pl.pallas_call(
        _loss_body,
        grid=(BQ // RB,),
        in_specs=[
            pl.BlockSpec((RB, D_MODEL), lambda i: (qb + i, 0)),
            pl.BlockSpec((RB, D_MODEL), lambda i: (i, 0)),
        ],
        out_specs=[
            pl.BlockSpec((RB, D_MODEL), lambda i: (i, 0)),
            pl.BlockSpec((RB, 1), lambda i: (i, 0)),
            pl.BlockSpec((RB, 1), lambda i: (i, 0)),
        ],
        out_shape=[
            jax.ShapeDtypeStruct((BQ, D_MODEL), jnp.float32),
            jax.ShapeDtypeStruct((BQ, 1), jnp.float32),
            jax.ShapeDtypeStruct((BQ, 1), jnp.float32),
        ],
    )(xn, yn)


def kernel(x, b_pre, W_enc, W_dec, b_post):
    xn, xm = _prep(x, b_pre.reshape(1, D_MODEL))
    av, aj = _encode_binmax(xm, W_enc)
    ws, iss, yns, ys, rls, lls = [], [], [], [], [], []
    for qi in range(NQ):
        w_q, i_q = _topk(av, aj, qi)
        yn_q = _decode(i_q, w_q, W_dec, b_post)
        ws.append(w_q)
        iss.append(i_q)
        yns.append(yn_q)
    for qi in range(NQ):
        y_q, rl_q, ll_q = _losses(xn, yns[qi], qi)
        ys.append(y_q)
        rls.append(rl_q)
        lls.append(ll_q)
    weights = jnp.concatenate(ws, axis=0)
    indices = jnp.concatenate(iss, axis=0)
    y_normed = jnp.concatenate(yns, axis=0)
    y = jnp.concatenate(ys, axis=0)
    recon_loss = jnp.concatenate(rls, axis=0).reshape(BATCH)
    loss = jnp.concatenate(lls, axis=0).reshape(BATCH)
    return (xn, x, weights, indices, y_normed, y, recon_loss, recon_loss, loss)
